# baseline (speedup 1.0000x reference)
import functools

import jax
import jax.numpy as jnp
from jax import lax
from jax.experimental import pallas as pl
from jax.experimental.pallas import tpu as pltpu
from jax.experimental.pallas import tpu_sc as plsc

_LN2 = 0.6931471805599453
_FMAX = 3.4028235e38


def _neg_log(v):
    bits = lax.bitcast_convert_type(v, jnp.int32)
    e = ((bits >> 23) & 0xFF) - 127
    mb = (bits & 0x007FFFFF) | 0x3F800000
    m = lax.bitcast_convert_type(mb, jnp.float32)
    big = m > 1.4142135
    m = jnp.where(big, m * 0.5, m)
    ef = e.astype(jnp.float32) + jnp.where(big, 1.0, 0.0)
    z = (m - 1.0) / (m + 1.0)
    z2 = z * z
    p = 1.0 + z2 * (1.0 / 3.0 + z2 * (0.2 + z2 * (1.0 / 7.0 + z2 * (1.0 / 9.0))))
    return -(ef * _LN2 + 2.0 * z * p)


def _compact_mask(grids_t, H, W, B):
    def body(g_hbm, o_ref, gbuf, sem):
        cp = pltpu.make_async_copy(
            g_hbm.at[H, :, :, pl.ds(W, 1), :], gbuf, sem
        )
        cp.start()
        cp.wait()
        o_ref[...] = gbuf[:, :, 0, :].astype(jnp.float32)

    return pl.pallas_call(
        body,
        grid=(1,),
        in_specs=[pl.BlockSpec(memory_space=pl.ANY)],
        out_specs=pl.BlockSpec((H, W, B), lambda i: (0, 0, 0)),
        out_shape=jax.ShapeDtypeStruct((H, W, B), jnp.float32),
        scratch_shapes=[
            pltpu.VMEM((H, W, 1, B), jnp.int32),
            pltpu.SemaphoreType.DMA,
        ],
    )(grids_t)


def kernel(predictions, image_grids, target_boxes_grid):
    B, H, W = predictions.shape
    HW = H * W
    x3 = jnp.transpose(predictions, (1, 2, 0))
    grids_t = jnp.transpose(image_grids, (0, 3, 4, 1, 2))
    m3 = _compact_mask(grids_t, H, W, B)

    L = 16
    LG = 128
    NK = LG // L
    NPG = 4
    APG = 4

    mesh = plsc.VectorSubcoreMesh(core_axis_name="c", subcore_axis_name="s")

    @functools.partial(
        pl.kernel,
        mesh=mesh,
        out_type=jax.ShapeDtypeStruct((16, 128), jnp.float32),
        scratch_types=[
            pltpu.VMEM((APG, W, LG), jnp.float32),
            pltpu.VMEM((APG, W, LG), jnp.float32),
            pltpu.VMEM((8, LG), jnp.float32),
            pltpu.VMEM_SHARED((16, 8, LG), jnp.float32),
            pltpu.VMEM((NPG, 8, LG), jnp.float32),
            pltpu.VMEM_SHARED((8, 128), jnp.float32),
            pltpu.VMEM((8, 128), jnp.float32),
            pltpu.VMEM((L,), jnp.float32),
            pltpu.VMEM((8, 128), jnp.float32),
        ],
    )
    def sck(x_hbm, m_hbm, out_hbm, xv, mv, part, shared, red, shared2, red2, outv, outb):
        cid = lax.axis_index("c")
        sid = lax.axis_index("s")
        pg = sid // NPG
        lg = sid % NPG
        lane0 = pl.multiple_of((cid * NPG + lg) * LG, LG)
        a0 = pg * APG
        na = jnp.where(pg < NPG - 1, APG, H - (NPG - 1) * APG)

        @pl.when(pg < NPG - 1)
        def _():
            pltpu.sync_copy(x_hbm.at[pl.ds(a0, APG), :, pl.ds(lane0, LG)], xv)
            pltpu.sync_copy(m_hbm.at[pl.ds(a0, APG), :, pl.ds(lane0, LG)], mv)

        @pl.when(pg == NPG - 1)
        def _():
            nl = H - (NPG - 1) * APG
            src = pl.ds((NPG - 1) * APG, nl)
            pltpu.sync_copy(x_hbm.at[src, :, pl.ds(lane0, LG)], xv.at[pl.ds(0, nl)])
            pltpu.sync_copy(m_hbm.at[src, :, pl.ds(lane0, LG)], mv.at[pl.ds(0, nl)])

        zeros = jnp.zeros((L,), jnp.float32)

        def body(a, carry):
            nxt = []
            for k in range(NK):
                s_pm, cnt, s_p = carry[k]
                for b in range(W):
                    x = xv[a, b, pl.ds(k * L, L)]
                    mm = mv[a, b, pl.ds(k * L, L)]
                    s_pm = s_pm + x * mm
                    cnt = cnt + mm
                    s_p = s_p + x
                nxt.append((s_pm, cnt, s_p))
            return tuple(nxt)

        accs = lax.fori_loop(
            0, na, body, tuple((zeros, zeros, zeros) for _ in range(NK))
        )
        for k in range(NK):
            part[0, pl.ds(k * L, L)] = accs[k][0]
            part[1, pl.ds(k * L, L)] = accs[k][1]
            part[2, pl.ds(k * L, L)] = accs[k][2]
        pltpu.sync_copy(part, shared.at[sid])
        plsc.subcore_barrier()

        @pl.when(sid < NPG)
        def _():
            for kk in range(NPG):
                pltpu.sync_copy(shared.at[kk * NPG + sid], red.at[kk])
            contrib = zeros
            for k in range(NK):
                s_pm = zeros
                cnt = zeros
                s_p = zeros
                for kk in range(NPG):
                    s_pm = s_pm + red[kk, 0, pl.ds(k * L, L)]
                    cnt = cnt + red[kk, 1, pl.ds(k * L, L)]
                    s_p = s_p + red[kk, 2, pl.ds(k * L, L)]
                mean_t = s_pm / cnt
                lt = jnp.where(mean_t > 0.0, _neg_log(mean_t), jnp.inf)
                mean_b = (s_p - s_pm) / (float(HW) - cnt)
                arg = 1.0 - mean_b
                lb = jnp.where(
                    arg > 0.0, _neg_log(arg), jnp.where(arg == 0.0, _FMAX, 0.0)
                )
                contrib = contrib + lt + lb
            outv[...] = contrib * (1.0 / B)
            pltpu.sync_copy(outv, shared2.at[sid, pl.ds(0, L)])

        plsc.subcore_barrier()

        @pl.when(sid == 0)
        def _():
            pltpu.sync_copy(shared2, red2)
            tot = (
                red2[0, pl.ds(0, L)]
                + red2[1, pl.ds(0, L)]
                + red2[2, pl.ds(0, L)]
                + red2[3, pl.ds(0, L)]
            )
            n = L
            while n > 1:
                n //= 2
                idx = (lax.iota(jnp.int32, L) + n) % L
                rot = lax.gather(
                    tot,
                    idx[:, None],
                    lax.GatherDimensionNumbers(
                        offset_dims=(),
                        collapsed_slice_dims=(0,),
                        start_index_map=(0,),
                    ),
                    slice_sizes=(1,),
                    mode=lax.GatherScatterMode.PROMISE_IN_BOUNDS,
                )
                tot = tot + rot
            outb[0, pl.ds(0, L)] = tot
            row0 = pl.multiple_of(cid * 8, 8)
            pltpu.sync_copy(outb, out_hbm.at[pl.ds(row0, 8), :])

    out = sck(x3, m3)
    return out[0, 0] + out[8, 0]
